# Initial kernel scaffold; baseline (speedup 1.0000x reference)
import jax
import jax.numpy as jnp
from jax.experimental import pallas as pl


def kernel(x, edge_index, edge_attr, u, batch, ew1, eb1, ew2, eb2, n1w1, n1b1, n1w2, n1b2, n2w1, n2b1, n2w2, n2b2, gw1, gb1, gw2, gb2):
    raise NotImplementedError("write your pallas kernel here")


# speedup vs baseline: 5.1838x; 5.1838x over previous
import functools

import jax
import jax.numpy as jnp
from jax import lax
from jax.experimental import pallas as pl
from jax.experimental.pallas import tpu as pltpu
from jax.experimental.pallas import tpu_sc as plsc

F32 = jnp.float32

NC = 2
NS = 16
NW = NC * NS
LANES = 16

T_EDGE = 128
T_NODE = 80


def _k1_body(x_ref, b16_ref, u_ref, wxr_ref, wxc_ref, wu_ref, eb1_ref,
             a_ref, c_ref):
    x = x_ref[...]
    u2 = jnp.dot(u_ref[...], wu_ref[...], preferred_element_type=F32)
    tn = b16_ref.shape[0]
    iota = lax.broadcasted_iota(jnp.int32, (tn, 16), 1)
    oh = (b16_ref[...] == iota).astype(F32)
    a_ref[...] = (jnp.dot(x, wxr_ref[...], preferred_element_type=F32)
                  + jnp.dot(oh, u2, preferred_element_type=F32)
                  + eb1_ref[...])
    c_ref[...] = jnp.dot(x, wxc_ref[...], preferred_element_type=F32)


def _run_k1(x, b16, u, wxr, wxc, wu, eb1, n, dn, tn):
    nblk = n // tn
    full = lambda *shape: pl.BlockSpec(shape, lambda i: tuple(0 for _ in shape))
    return pl.pallas_call(
        _k1_body,
        grid=(nblk,),
        in_specs=[
            pl.BlockSpec((tn, dn), lambda i: (i, 0)),
            pl.BlockSpec((tn, 16), lambda i: (i, 0)),
            full(16, 32),
            full(dn, 128),
            full(dn, 128),
            full(32, 128),
            full(1, 128),
        ],
        out_specs=[
            pl.BlockSpec((tn, 128), lambda i: (i, 0)),
            pl.BlockSpec((tn, 128), lambda i: (i, 0)),
        ],
        out_shape=[
            jax.ShapeDtypeStruct((n, 128), F32),
            jax.ShapeDtypeStruct((n, 128), F32),
        ],
    )(x, b16, u, wxr, wxc, wu, eb1)


def _k2_body(n, e, a_hbm, c_hbm, x_hbm, row_hbm, col_hbm,
             g_hbm, aggx_hbm,
             idx_r, idx_c, buf_a, buf_c, buf_x, aggx_sh, s1, s2, s3):
    cid = lax.axis_index("c")
    sid = lax.axis_index("s")
    wid = sid * NC + cid
    nchunks_tot = e // T_EDGE
    nchunks_n = n // T_NODE

    zv = jnp.zeros((LANES,), F32)

    def zrow(i, carry):
        for j in range(128 // LANES):
            buf_a[i, pl.ds(j * LANES, LANES)] = zv
        return carry

    lax.fori_loop(0, T_EDGE, zrow, 0)
    nz_w = (nchunks_n - sid + NS - 1) // NS

    def zchunk(ci, carry):
        b = pl.multiple_of((sid + ci * NS) * T_NODE, T_NODE)
        pltpu.sync_copy(buf_a.at[pl.ds(0, T_NODE)],
                        aggx_sh.at[pl.ds(b, T_NODE)])
        return carry

    lax.fori_loop(0, nz_w, zchunk, 0)
    plsc.subcore_barrier()

    nchunks_w = (nchunks_tot - wid + NW - 1) // NW

    def chunk(ci, carry):
        c = wid + ci * NW
        base = pl.multiple_of(c * T_EDGE, T_EDGE)
        pltpu.sync_copy(row_hbm.at[pl.ds(base, T_EDGE)], idx_r)
        pltpu.sync_copy(col_hbm.at[pl.ds(base, T_EDGE)], idx_c)
        cp_a = pltpu.async_copy(a_hbm.at[idx_r], buf_a, s1)
        cp_c = pltpu.async_copy(c_hbm.at[idx_c], buf_c, s2)
        cp_x = pltpu.async_copy(x_hbm.at[idx_r], buf_x, s3)
        cp_a.wait()
        cp_c.wait()

        def addrow(i, cy):
            for j in range(128 // LANES):
                sl = pl.ds(j * LANES, LANES)
                buf_a[i, sl] = buf_a[i, sl] + buf_c[i, sl]
            return cy

        lax.fori_loop(0, T_EDGE, addrow, 0)
        pltpu.sync_copy(buf_a, g_hbm.at[pl.ds(base, T_EDGE)])
        cp_x.wait()
        pltpu.sync_copy(buf_x, aggx_sh.at[idx_c], add=True)
        return carry

    lax.fori_loop(0, nchunks_w, chunk, 0)
    plsc.subcore_barrier()

    def ochunk(ci, carry):
        b = pl.multiple_of((sid + ci * NS) * T_NODE, T_NODE)
        ob = pl.multiple_of(cid * n + (sid + ci * NS) * T_NODE, T_NODE)
        pltpu.sync_copy(aggx_sh.at[pl.ds(b, T_NODE)],
                        aggx_hbm.at[pl.ds(ob, T_NODE)])
        return carry

    lax.fori_loop(0, nz_w, ochunk, 0)


def _run_k2(a, c, x, row, col, n, e):
    mesh = plsc.VectorSubcoreMesh(core_axis_name="c", subcore_axis_name="s",
                                  num_cores=NC, num_subcores=NS)
    k = functools.partial(
        pl.kernel,
        out_type=(jax.ShapeDtypeStruct((e, 128), F32),
                  jax.ShapeDtypeStruct((NC * n, 128), F32)),
        mesh=mesh,
        scratch_types=[
            pltpu.VMEM((T_EDGE,), jnp.int32),
            pltpu.VMEM((T_EDGE,), jnp.int32),
            pltpu.VMEM((T_EDGE, 128), F32),
            pltpu.VMEM((T_EDGE, 128), F32),
            pltpu.VMEM((T_EDGE, 128), F32),
            pltpu.VMEM_SHARED((n, 128), F32),
            pltpu.SemaphoreType.DMA,
            pltpu.SemaphoreType.DMA,
            pltpu.SemaphoreType.DMA,
        ],
    )(functools.partial(_k2_body, n, e))
    return k(a, c, x, row, col)


def _k3_body(g_ref, ea_ref, we_ref, ew2_ref, eb2_ref, e_ref):
    eh = jnp.maximum(
        g_ref[...] + jnp.dot(ea_ref[...], we_ref[...],
                             preferred_element_type=F32), 0.0)
    e_ref[...] = (jnp.dot(eh, ew2_ref[...], preferred_element_type=F32)
                  + eb2_ref[...])


def _run_k3(g, ea, we, ew2, eb2, e, de, te):
    nblk = e // te
    full = lambda *shape: pl.BlockSpec(shape, lambda i: tuple(0 for _ in shape))
    return pl.pallas_call(
        _k3_body,
        grid=(nblk,),
        in_specs=[
            pl.BlockSpec((te, 128), lambda i: (i, 0)),
            pl.BlockSpec((te, de), lambda i: (i, 0)),
            full(de, 128),
            full(128, de),
            full(1, de),
        ],
        out_specs=pl.BlockSpec((te, de), lambda i: (i, 0)),
        out_shape=jax.ShapeDtypeStruct((e, de), F32),
    )(g, ea, we, ew2, eb2)


def _k4_body(n, e, ev_hbm, row_hbm, col_hbm,
             acc_ce_hbm, acc_re_hbm,
             idx, buf_e, buf_s, acc_sh):
    cid = lax.axis_index("c")
    sid = lax.axis_index("s")
    wid = sid * NC + cid
    nchunks_tot = e // T_EDGE
    nchunks_n = n // T_NODE
    rows_per_chunk = T_EDGE // 8

    zv = jnp.zeros((LANES,), F32)
    lane = lax.iota(jnp.int32, LANES)
    onev = jnp.where(lane == 0, 1.0, 0.0).astype(F32)

    nz_w = (nchunks_n - sid + NS - 1) // NS
    nchunks_w = (nchunks_tot - wid + NW - 1) // NW

    for phase, (idx_hbm, out_hbm, markv) in enumerate(
            ((col_hbm, acc_ce_hbm, zv), (row_hbm, acc_re_hbm, onev))):
        def zrow(i, carry):
            for j in range(128 // LANES):
                buf_s[i, pl.ds(j * LANES, LANES)] = zv
            return carry

        lax.fori_loop(0, T_EDGE, zrow, 0)

        def zchunk(ci, carry):
            b = pl.multiple_of((sid + ci * NS) * T_NODE, T_NODE)
            pltpu.sync_copy(buf_s.at[pl.ds(0, T_NODE)],
                            acc_sh.at[pl.ds(b, T_NODE)])
            return carry

        lax.fori_loop(0, nz_w, zchunk, 0)
        plsc.subcore_barrier()

        def chunk(ci, carry):
            c = wid + ci * NW
            base = pl.multiple_of(c * T_EDGE, T_EDGE)
            vbase = pl.multiple_of(c * rows_per_chunk, rows_per_chunk)
            pltpu.sync_copy(idx_hbm.at[pl.ds(base, T_EDGE)], idx)
            pltpu.sync_copy(ev_hbm.at[pl.ds(vbase, rows_per_chunk)], buf_e)

            def build(i, cy):
                for k in range(8):
                    v = buf_e[i, pl.ds(k * LANES, LANES)]
                    buf_s[i * 8 + k, pl.ds(0, LANES)] = v
                    buf_s[i * 8 + k, pl.ds(LANES, LANES)] = markv
                return cy

            lax.fori_loop(0, rows_per_chunk, build, 0)
            pltpu.sync_copy(buf_s, acc_sh.at[idx], add=True)
            return carry

        lax.fori_loop(0, nchunks_w, chunk, 0)
        plsc.subcore_barrier()

        def ochunk(ci, carry):
            b = pl.multiple_of((sid + ci * NS) * T_NODE, T_NODE)
            ob = pl.multiple_of(cid * n + (sid + ci * NS) * T_NODE, T_NODE)
            pltpu.sync_copy(acc_sh.at[pl.ds(b, T_NODE)],
                            out_hbm.at[pl.ds(ob, T_NODE)])
            return carry

        lax.fori_loop(0, nz_w, ochunk, 0)
        if phase == 0:
            plsc.subcore_barrier()


def _run_k4(e_view, row, col, n, e):
    mesh = plsc.VectorSubcoreMesh(core_axis_name="c", subcore_axis_name="s",
                                  num_cores=NC, num_subcores=NS)
    sds = jax.ShapeDtypeStruct((NC * n, 128), F32)
    k = functools.partial(
        pl.kernel,
        out_type=(sds, sds),
        mesh=mesh,
        scratch_types=[
            pltpu.VMEM((T_EDGE,), jnp.int32),
            pltpu.VMEM((T_EDGE // 8, 128), F32),
            pltpu.VMEM((T_EDGE, 128), F32),
            pltpu.VMEM_SHARED((n, 128), F32),
        ],
    )(functools.partial(_k4_body, n, e))
    return k(e_view, row, col)


def _k5_body(nblk,
             x_ref, b16_ref, u_ref,
             aggx0, aggx1, ce0, ce1, re0, re1,
             n1w1x, n1w1e, n1b1, n1w2, n1b2,
             n2w1x, n2w1h, n2w1u, n2b1, n2w2, n2b2,
             gw1u, gw1n, gw1e, gb1, gw2, gb2,
             xn_ref, un_ref,
             s_nsum, s_esum, s_ecnt, s_ncnt):
    i = pl.program_id(0)
    tn = x_ref.shape[0]
    iota = lax.broadcasted_iota(jnp.int32, (tn, 16), 1)
    oh = (b16_ref[...] == iota).astype(F32)

    aggx = aggx0[...] + aggx1[...]
    acc_ce = ce0[...] + ce1[...]
    acc_re = re0[...] + re1[...]
    aggec = acc_ce[:, 0:16]
    agger = acc_re[:, 0:16]
    cntcol = acc_re[:, 16:32]

    s = (jnp.dot(aggx, n1w1x[...], preferred_element_type=F32)
         + jnp.dot(aggec, n1w1e[...], preferred_element_type=F32)
         + n1b1[...])
    h = (jnp.dot(jnp.maximum(s, 0.0), n1w2[...], preferred_element_type=F32)
         + n1b2[...])
    u3 = jnp.dot(u_ref[...], n2w1u[...], preferred_element_type=F32)
    pre = (jnp.dot(x_ref[...], n2w1x[...], preferred_element_type=F32)
           + jnp.dot(h, n2w1h[...], preferred_element_type=F32)
           + jnp.dot(oh, u3, preferred_element_type=F32)
           + n2b1[...])
    xn = (jnp.dot(jnp.maximum(pre, 0.0), n2w2[...],
                  preferred_element_type=F32) + n2b2[...])
    xn_ref[...] = xn

    @pl.when(i == 0)
    def _():
        s_nsum[...] = jnp.zeros_like(s_nsum)
        s_esum[...] = jnp.zeros_like(s_esum)
        s_ecnt[...] = jnp.zeros_like(s_ecnt)
        s_ncnt[...] = jnp.zeros_like(s_ncnt)

    dnums = (((0,), (0,)), ((), ()))
    s_nsum[...] += lax.dot_general(oh, xn, dnums, preferred_element_type=F32)
    s_esum[...] += lax.dot_general(oh, agger, dnums,
                                   preferred_element_type=F32)
    s_ecnt[...] += lax.dot_general(oh, cntcol, dnums,
                                   preferred_element_type=F32)
    s_ncnt[...] += lax.dot_general(oh, jnp.ones_like(oh), dnums,
                                   preferred_element_type=F32)

    @pl.when(i == nblk - 1)
    def _():
        ncnt = jnp.maximum(s_ncnt[:, 0:1], 1.0)
        node_info = s_nsum[...] / ncnt
        ecnt = jnp.maximum(s_ecnt[:, 0:1], 1.0)
        edge_info = s_esum[...] / ecnt
        g1 = (jnp.dot(u_ref[...], gw1u[...], preferred_element_type=F32)
              + jnp.dot(node_info, gw1n[...], preferred_element_type=F32)
              + jnp.dot(edge_info, gw1e[...], preferred_element_type=F32)
              + gb1[...])
        un_ref[...] = (jnp.dot(jnp.maximum(g1, 0.0), gw2[...],
                               preferred_element_type=F32) + gb2[...])


def _run_k5(x, b16, u, aggx_pp, acc_ce_pp, acc_re_pp,
            n1w1x, n1w1e, n1b1, n1w2, n1b2,
            n2w1x, n2w1h, n2w1u, n2b1, n2w2, n2b2,
            gw1u, gw1n, gw1e, gb1, gw2, gb2,
            n, dn, dg, de, tn):
    nblk = n // tn
    full = lambda *shape: pl.BlockSpec(shape, lambda i: tuple(0 for _ in shape))
    p0 = pl.BlockSpec((tn, 128), lambda i: (i, 0))
    p1 = pl.BlockSpec((tn, 128), lambda i: (i + nblk, 0))
    return pl.pallas_call(
        functools.partial(_k5_body, nblk),
        grid=(nblk,),
        in_specs=[
            pl.BlockSpec((tn, dn), lambda i: (i, 0)),
            pl.BlockSpec((tn, 16), lambda i: (i, 0)),
            full(16, dg),
            p0, p1, p0, p1, p0, p1,
            full(dn, 128), full(de, 128), full(1, 128),
            full(128, 128), full(1, 128),
            full(dn, 128), full(128, 128), full(dg, 128), full(1, 128),
            full(128, dn), full(1, dn),
            full(dg, 128), full(dn, 128), full(de, 128), full(1, 128),
            full(128, dg), full(1, dg),
        ],
        out_specs=[
            pl.BlockSpec((tn, dn), lambda i: (i, 0)),
            pl.BlockSpec((16, dg), lambda i: (0, 0)),
        ],
        out_shape=[
            jax.ShapeDtypeStruct((n, dn), F32),
            jax.ShapeDtypeStruct((16, dg), F32),
        ],
        scratch_shapes=[
            pltpu.VMEM((16, 128), F32),
            pltpu.VMEM((16, de), F32),
            pltpu.VMEM((16, 16), F32),
            pltpu.VMEM((16, 16), F32),
        ],
    )(x, b16, u, aggx_pp, aggx_pp, acc_ce_pp, acc_ce_pp,
      acc_re_pp, acc_re_pp,
      n1w1x, n1w1e, n1b1, n1w2, n1b2,
      n2w1x, n2w1h, n2w1u, n2b1, n2w2, n2b2,
      gw1u, gw1n, gw1e, gb1, gw2, gb2)


def kernel(x, edge_index, edge_attr, u, batch,
           ew1, eb1, ew2, eb2,
           n1w1, n1b1, n1w2, n1b2,
           n2w1, n2b1, n2w2, n2b2,
           gw1, gb1, gw2, gb2):
    n, dn = x.shape
    e, de = edge_attr.shape
    dg = u.shape[1]
    tn = 1000

    row = edge_index[0]
    col = edge_index[1]
    b16 = jnp.broadcast_to(batch[:, None], (n, 16))

    wxr = ew1[0:dn]
    wxc = ew1[dn:2 * dn]
    we = ew1[2 * dn:2 * dn + de]
    wu = ew1[2 * dn + de:]
    eb1r = eb1[None, :]
    eb2r = eb2[None, :]
    n1w1x = n1w1[0:dn]
    n1w1e = n1w1[dn:]
    n2w1x = n2w1[0:dn]
    n2w1h = n2w1[dn:dn + 128]
    n2w1u = n2w1[dn + 128:]
    gw1u = gw1[0:dg]
    gw1n = gw1[dg:dg + dn]
    gw1e = gw1[dg + dn:]

    a, c = _run_k1(x, b16, u, wxr, wxc, wu, eb1r, n, dn, tn)
    g, aggx_pp = _run_k2(a, c, x, row, col, n, e)
    e_new = _run_k3(g, edge_attr, we, ew2, eb2r, e, de, 2000)
    e_view = jnp.reshape(e_new, (e // 8, 128))
    acc_ce_pp, acc_re_pp = _run_k4(e_view, row, col, n, e)
    x_new, u_new = _run_k5(
        x, b16, u, aggx_pp, acc_ce_pp, acc_re_pp,
        n1w1x, n1w1e, n1b1[None, :], n1w2, n1b2[None, :],
        n2w1x, n2w1h, n2w1u, n2b1[None, :], n2w2, n2b2[None, :],
        gw1u, gw1n, gw1e, gb1[None, :], gw2, gb2[None, :],
        n, dn, dg, de, tn)
    return (x_new, e_new, u_new)


# speedup vs baseline: 6.5739x; 1.2682x over previous
import functools

import jax
import jax.numpy as jnp
from jax import lax
from jax.experimental import pallas as pl
from jax.experimental.pallas import tpu as pltpu
from jax.experimental.pallas import tpu_sc as plsc

F32 = jnp.float32

NC = 2
NS = 16
NW = NC * NS
LANES = 16

T_EDGE = 128
T_NODE = 80


def _k1_body(x_ref, b16_ref, u_ref, wxr_ref, wxc_ref, wu_ref, eb1_ref,
             a_ref, c_ref):
    x = x_ref[...]
    u2 = jnp.dot(u_ref[...], wu_ref[...], preferred_element_type=F32)
    tn = b16_ref.shape[0]
    iota = lax.broadcasted_iota(jnp.int32, (tn, 16), 1)
    oh = (b16_ref[...] == iota).astype(F32)
    a_ref[...] = (jnp.dot(x, wxr_ref[...], preferred_element_type=F32)
                  + jnp.dot(oh, u2, preferred_element_type=F32)
                  + eb1_ref[...])
    c_ref[...] = jnp.dot(x, wxc_ref[...], preferred_element_type=F32)


def _run_k1(x, b16, u, wxr, wxc, wu, eb1, n, dn, tn):
    nblk = n // tn
    full = lambda *shape: pl.BlockSpec(shape, lambda i: tuple(0 for _ in shape))
    return pl.pallas_call(
        _k1_body,
        grid=(nblk,),
        in_specs=[
            pl.BlockSpec((tn, dn), lambda i: (i, 0)),
            pl.BlockSpec((tn, 16), lambda i: (i, 0)),
            full(16, 32),
            full(dn, 128),
            full(dn, 128),
            full(32, 128),
            full(1, 128),
        ],
        out_specs=[
            pl.BlockSpec((tn, 128), lambda i: (i, 0)),
            pl.BlockSpec((tn, 128), lambda i: (i, 0)),
        ],
        out_shape=[
            jax.ShapeDtypeStruct((n, 128), F32),
            jax.ShapeDtypeStruct((n, 128), F32),
        ],
    )(x, b16, u, wxr, wxc, wu, eb1)


def _k2_body(n, e, a_hbm, c_hbm, row_hbm, col_hbm, g_hbm,
             idx_r, idx_c, buf_a, buf_c,
             sa0, sa1, sc0, sc1, sg0, sg1):
    cid = lax.axis_index("c")
    sid = lax.axis_index("s")
    wid = sid * NC + cid
    nchunks_tot = e // T_EDGE
    sa = (sa0, sa1)
    sc = (sc0, sc1)
    sg = (sg0, sg1)

    def load_idx(cc, slot):
        base = pl.multiple_of(cc * T_EDGE, T_EDGE)
        pltpu.sync_copy(row_hbm.at[pl.ds(base, T_EDGE)], idx_r.at[slot])
        pltpu.sync_copy(col_hbm.at[pl.ds(base, T_EDGE)], idx_c.at[slot])

    def fire_gathers(slot):
        pltpu.async_copy(a_hbm.at[idx_r.at[slot]], buf_a.at[slot], sa[slot])
        pltpu.async_copy(c_hbm.at[idx_c.at[slot]], buf_c.at[slot], sc[slot])

    def wait_gathers(slot):
        pltpu.make_async_copy(a_hbm.at[idx_r.at[slot]], buf_a.at[slot],
                              sa[slot]).wait()
        pltpu.make_async_copy(c_hbm.at[idx_c.at[slot]], buf_c.at[slot],
                              sc[slot]).wait()

    def wait_gwrite(cc, slot):
        base = pl.multiple_of(cc * T_EDGE, T_EDGE)
        pltpu.make_async_copy(buf_a.at[slot], g_hbm.at[pl.ds(base, T_EDGE)],
                              sg[slot]).wait()

    load_idx(wid, 0)
    fire_gathers(0)

    kmax = (nchunks_tot + NW - 1) // NW
    niter = (kmax + 1) // 2

    def step(ci, carry):
        for b in (0, 1):
            k = ci * 2 + b
            c = wid + k * NW
            cn = c + NW
            nb = 1 - b

            @pl.when(cn < nchunks_tot)
            def _():
                @pl.when(k >= 1)
                def _():
                    wait_gwrite(c - NW, nb)

                load_idx(cn, nb)
                fire_gathers(nb)

            @pl.when(c < nchunks_tot)
            def _():
                wait_gathers(b)
                base = pl.multiple_of(c * T_EDGE, T_EDGE)

                def addrow(i, cy):
                    for j in range(128 // LANES):
                        sl = pl.ds(j * LANES, LANES)
                        buf_a[b, i, sl] = buf_a[b, i, sl] + buf_c[b, i, sl]
                    return cy

                lax.fori_loop(0, T_EDGE, addrow, 0)
                pltpu.async_copy(buf_a.at[b], g_hbm.at[pl.ds(base, T_EDGE)],
                                 sg[b])
        return carry

    lax.fori_loop(0, niter, step, 0)
    nchunks_w = (nchunks_tot - wid + NW - 1) // NW
    for b in (0, 1):
        kb = nchunks_w - 1 - ((nchunks_w - 1 - b) % 2)
        wait_gwrite(wid + kb * NW, b)


def _run_k2(a, c, row, col, n, e):
    mesh = plsc.VectorSubcoreMesh(core_axis_name="c", subcore_axis_name="s",
                                  num_cores=NC, num_subcores=NS)
    k = functools.partial(
        pl.kernel,
        out_type=jax.ShapeDtypeStruct((e, 128), F32),
        mesh=mesh,
        scratch_types=[
            pltpu.VMEM((2, T_EDGE), jnp.int32),
            pltpu.VMEM((2, T_EDGE), jnp.int32),
            pltpu.VMEM((2, T_EDGE, 128), F32),
            pltpu.VMEM((2, T_EDGE, 128), F32),
            pltpu.SemaphoreType.DMA,
            pltpu.SemaphoreType.DMA,
            pltpu.SemaphoreType.DMA,
            pltpu.SemaphoreType.DMA,
            pltpu.SemaphoreType.DMA,
            pltpu.SemaphoreType.DMA,
        ],
    )(functools.partial(_k2_body, n, e))
    return k(a, c, row, col)


def _k3_body(g_ref, ea_ref, we_ref, ew2_ref, eb2_ref, e_ref):
    eh = jnp.maximum(
        g_ref[...] + jnp.dot(ea_ref[...], we_ref[...],
                             preferred_element_type=F32), 0.0)
    e_ref[...] = (jnp.dot(eh, ew2_ref[...], preferred_element_type=F32)
                  + eb2_ref[...])


def _run_k3(g, ea, we, ew2, eb2, e, de, te):
    nblk = e // te
    full = lambda *shape: pl.BlockSpec(shape, lambda i: tuple(0 for _ in shape))
    return pl.pallas_call(
        _k3_body,
        grid=(nblk,),
        in_specs=[
            pl.BlockSpec((te, 128), lambda i: (i, 0)),
            pl.BlockSpec((te, de), lambda i: (i, 0)),
            full(de, 128),
            full(128, de),
            full(1, de),
        ],
        out_specs=pl.BlockSpec((te, de), lambda i: (i, 0)),
        out_shape=jax.ShapeDtypeStruct((e, de), F32),
    )(g, ea, we, ew2, eb2)


def _k4_body(n, e, x_hbm, ev_hbm, row_hbm, col_hbm,
             aggx_hbm, acc_ce_hbm, acc_re_hbm,
             idxa, idxb, buf_e, buf_s, zbuf, acc_sh,
             si0, si1, ss0, ss1):
    cid = lax.axis_index("c")
    sid = lax.axis_index("s")
    wid = sid * NC + cid
    nchunks_tot = e // T_EDGE
    nchunks_n = n // T_NODE
    erows = T_EDGE // 8
    si = (si0, si1)
    ss = (ss0, ss1)

    zv = jnp.zeros((LANES,), F32)
    lane = lax.iota(jnp.int32, LANES)
    onev = jnp.where(lane == 0, 1.0, 0.0).astype(F32)

    nz_w = (nchunks_n - sid + NS - 1) // NS
    kmax = (nchunks_tot + NW - 1) // NW
    niter = (kmax + 1) // 2
    nchunks_w = (nchunks_tot - wid + NW - 1) // NW

    def z0(i, carry):
        for j in range(128 // LANES):
            zbuf[i, pl.ds(j * LANES, LANES)] = zv
        return carry

    lax.fori_loop(0, T_NODE, z0, 0)

    def z1(i, carry):
        for b in (0, 1):
            for j in range(128 // LANES):
                buf_s[b, i, pl.ds(j * LANES, LANES)] = zv
        return carry

    lax.fori_loop(0, T_EDGE, z1, 0)

    def zero_acc():
        def zchunk(ci, carry):
            b = pl.multiple_of((sid + ci * NS) * T_NODE, T_NODE)
            pltpu.sync_copy(zbuf, acc_sh.at[pl.ds(b, T_NODE)])
            return carry

        lax.fori_loop(0, nz_w, zchunk, 0)

    def copy_out(out_hbm):
        def ochunk(ci, carry):
            b = pl.multiple_of((sid + ci * NS) * T_NODE, T_NODE)
            ob = pl.multiple_of(cid * n + (sid + ci * NS) * T_NODE, T_NODE)
            pltpu.sync_copy(acc_sh.at[pl.ds(b, T_NODE)],
                            out_hbm.at[pl.ds(ob, T_NODE)])
            return carry

        lax.fori_loop(0, nz_w, ochunk, 0)

    def wait_scatter(slot, sidx_ref):
        pltpu.make_async_copy(buf_s.at[slot], acc_sh.at[sidx_ref.at[slot]],
                              ss[slot]).wait()

    def drain_tail(sidx_ref):
        for b in (0, 1):
            wait_scatter(b, sidx_ref)

    zero_acc()
    plsc.subcore_barrier()

    def load_idx_x(cc, slot):
        base = pl.multiple_of(cc * T_EDGE, T_EDGE)
        pltpu.sync_copy(row_hbm.at[pl.ds(base, T_EDGE)], idxa.at[slot])
        pltpu.sync_copy(col_hbm.at[pl.ds(base, T_EDGE)], idxb.at[slot])

    load_idx_x(wid, 0)
    pltpu.async_copy(x_hbm.at[idxa.at[0]], buf_s.at[0], si[0])

    def step_x(ci, carry):
        for b in (0, 1):
            k = ci * 2 + b
            c = wid + k * NW
            cn = c + NW
            nb = 1 - b

            @pl.when(cn < nchunks_tot)
            def _():
                @pl.when(k >= 1)
                def _():
                    wait_scatter(nb, idxb)

                load_idx_x(cn, nb)
                pltpu.async_copy(x_hbm.at[idxa.at[nb]], buf_s.at[nb],
                                 si[nb])

            @pl.when(c < nchunks_tot)
            def _():
                pltpu.make_async_copy(x_hbm.at[idxa.at[b]], buf_s.at[b],
                                      si[b]).wait()
                pltpu.async_copy(buf_s.at[b], acc_sh.at[idxb.at[b]],
                                 ss[b], add=True)
        return carry

    lax.fori_loop(0, niter, step_x, 0)
    drain_tail(idxb)
    plsc.subcore_barrier()
    copy_out(aggx_hbm)
    plsc.subcore_barrier()

    lax.fori_loop(0, T_EDGE, z1, 0)

    for phase, (idx_hbm, out_hbm, markv) in enumerate(
            ((col_hbm, acc_ce_hbm, zv), (row_hbm, acc_re_hbm, onev))):
        zero_acc()
        plsc.subcore_barrier()

        def fire_reads(cc, slot):
            base = pl.multiple_of(cc * T_EDGE, T_EDGE)
            vbase = pl.multiple_of(cc * erows, erows)
            pltpu.async_copy(idx_hbm.at[pl.ds(base, T_EDGE)],
                             idxa.at[slot], si[slot])
            pltpu.async_copy(ev_hbm.at[pl.ds(vbase, erows)],
                             buf_e.at[slot], si[slot])

        def wait_reads(cc, slot):
            base = pl.multiple_of(cc * T_EDGE, T_EDGE)
            vbase = pl.multiple_of(cc * erows, erows)
            pltpu.make_async_copy(idx_hbm.at[pl.ds(base, T_EDGE)],
                                  idxa.at[slot], si[slot]).wait()
            pltpu.make_async_copy(ev_hbm.at[pl.ds(vbase, erows)],
                                  buf_e.at[slot], si[slot]).wait()

        fire_reads(wid, 0)

        def step(ci, carry):
            for b in (0, 1):
                k = ci * 2 + b
                c = wid + k * NW
                cn = c + NW
                nb = 1 - b

                @pl.when(cn < nchunks_tot)
                def _():
                    @pl.when(k >= 1)
                    def _():
                        wait_scatter(nb, idxa)

                    fire_reads(cn, nb)

                @pl.when(c < nchunks_tot)
                def _():
                    wait_reads(c, b)

                    def build(i, cy):
                        for j in range(8):
                            v = buf_e[b, i, pl.ds(j * LANES, LANES)]
                            buf_s[b, i * 8 + j, pl.ds(0, LANES)] = v
                            buf_s[b, i * 8 + j, pl.ds(LANES, LANES)] = markv
                        return cy

                    lax.fori_loop(0, erows, build, 0)
                    pltpu.async_copy(buf_s.at[b], acc_sh.at[idxa.at[b]],
                                     ss[b], add=True)
            return carry

        lax.fori_loop(0, niter, step, 0)
        drain_tail(idxa)
        plsc.subcore_barrier()
        copy_out(out_hbm)
        if phase == 0:
            plsc.subcore_barrier()


def _run_k4(x, e_view, row, col, n, e):
    mesh = plsc.VectorSubcoreMesh(core_axis_name="c", subcore_axis_name="s",
                                  num_cores=NC, num_subcores=NS)
    sds = jax.ShapeDtypeStruct((NC * n, 128), F32)
    k = functools.partial(
        pl.kernel,
        out_type=(sds, sds, sds),
        mesh=mesh,
        scratch_types=[
            pltpu.VMEM((2, T_EDGE), jnp.int32),
            pltpu.VMEM((2, T_EDGE), jnp.int32),
            pltpu.VMEM((2, T_EDGE // 8, 128), F32),
            pltpu.VMEM((2, T_EDGE, 128), F32),
            pltpu.VMEM((T_NODE, 128), F32),
            pltpu.VMEM_SHARED((n, 128), F32),
            pltpu.SemaphoreType.DMA,
            pltpu.SemaphoreType.DMA,
            pltpu.SemaphoreType.DMA,
            pltpu.SemaphoreType.DMA,
        ],
    )(functools.partial(_k4_body, n, e))
    return k(x, e_view, row, col)


def _k5_body(nblk,
             x_ref, b16_ref, u_ref,
             aggx0, aggx1, ce0, ce1, re0, re1,
             n1w1x, n1w1e, n1b1, n1w2, n1b2,
             n2w1x, n2w1h, n2w1u, n2b1, n2w2, n2b2,
             gw1u, gw1n, gw1e, gb1, gw2, gb2,
             xn_ref, un_ref,
             s_nsum, s_esum, s_ecnt, s_ncnt):
    i = pl.program_id(0)
    tn = x_ref.shape[0]
    iota = lax.broadcasted_iota(jnp.int32, (tn, 16), 1)
    oh = (b16_ref[...] == iota).astype(F32)

    aggx = aggx0[...] + aggx1[...]
    acc_ce = ce0[...] + ce1[...]
    acc_re = re0[...] + re1[...]
    aggec = acc_ce[:, 0:16]
    agger = acc_re[:, 0:16]
    cntcol = acc_re[:, 16:32]

    s = (jnp.dot(aggx, n1w1x[...], preferred_element_type=F32)
         + jnp.dot(aggec, n1w1e[...], preferred_element_type=F32)
         + n1b1[...])
    h = (jnp.dot(jnp.maximum(s, 0.0), n1w2[...], preferred_element_type=F32)
         + n1b2[...])
    u3 = jnp.dot(u_ref[...], n2w1u[...], preferred_element_type=F32)
    pre = (jnp.dot(x_ref[...], n2w1x[...], preferred_element_type=F32)
           + jnp.dot(h, n2w1h[...], preferred_element_type=F32)
           + jnp.dot(oh, u3, preferred_element_type=F32)
           + n2b1[...])
    xn = (jnp.dot(jnp.maximum(pre, 0.0), n2w2[...],
                  preferred_element_type=F32) + n2b2[...])
    xn_ref[...] = xn

    @pl.when(i == 0)
    def _():
        s_nsum[...] = jnp.zeros_like(s_nsum)
        s_esum[...] = jnp.zeros_like(s_esum)
        s_ecnt[...] = jnp.zeros_like(s_ecnt)
        s_ncnt[...] = jnp.zeros_like(s_ncnt)

    dnums = (((0,), (0,)), ((), ()))
    s_nsum[...] += lax.dot_general(oh, xn, dnums, preferred_element_type=F32)
    s_esum[...] += lax.dot_general(oh, agger, dnums,
                                   preferred_element_type=F32)
    s_ecnt[...] += lax.dot_general(oh, cntcol, dnums,
                                   preferred_element_type=F32)
    s_ncnt[...] += lax.dot_general(oh, jnp.ones_like(oh), dnums,
                                   preferred_element_type=F32)

    @pl.when(i == nblk - 1)
    def _():
        ncnt = jnp.maximum(s_ncnt[:, 0:1], 1.0)
        node_info = s_nsum[...] / ncnt
        ecnt = jnp.maximum(s_ecnt[:, 0:1], 1.0)
        edge_info = s_esum[...] / ecnt
        g1 = (jnp.dot(u_ref[...], gw1u[...], preferred_element_type=F32)
              + jnp.dot(node_info, gw1n[...], preferred_element_type=F32)
              + jnp.dot(edge_info, gw1e[...], preferred_element_type=F32)
              + gb1[...])
        un_ref[...] = (jnp.dot(jnp.maximum(g1, 0.0), gw2[...],
                               preferred_element_type=F32) + gb2[...])


def _run_k5(x, b16, u, aggx_pp, acc_ce_pp, acc_re_pp,
            n1w1x, n1w1e, n1b1, n1w2, n1b2,
            n2w1x, n2w1h, n2w1u, n2b1, n2w2, n2b2,
            gw1u, gw1n, gw1e, gb1, gw2, gb2,
            n, dn, dg, de, tn):
    nblk = n // tn
    full = lambda *shape: pl.BlockSpec(shape, lambda i: tuple(0 for _ in shape))
    p0 = pl.BlockSpec((tn, 128), lambda i: (i, 0))
    p1 = pl.BlockSpec((tn, 128), lambda i: (i + nblk, 0))
    return pl.pallas_call(
        functools.partial(_k5_body, nblk),
        grid=(nblk,),
        in_specs=[
            pl.BlockSpec((tn, dn), lambda i: (i, 0)),
            pl.BlockSpec((tn, 16), lambda i: (i, 0)),
            full(16, dg),
            p0, p1, p0, p1, p0, p1,
            full(dn, 128), full(de, 128), full(1, 128),
            full(128, 128), full(1, 128),
            full(dn, 128), full(128, 128), full(dg, 128), full(1, 128),
            full(128, dn), full(1, dn),
            full(dg, 128), full(dn, 128), full(de, 128), full(1, 128),
            full(128, dg), full(1, dg),
        ],
        out_specs=[
            pl.BlockSpec((tn, dn), lambda i: (i, 0)),
            pl.BlockSpec((16, dg), lambda i: (0, 0)),
        ],
        out_shape=[
            jax.ShapeDtypeStruct((n, dn), F32),
            jax.ShapeDtypeStruct((16, dg), F32),
        ],
        scratch_shapes=[
            pltpu.VMEM((16, 128), F32),
            pltpu.VMEM((16, de), F32),
            pltpu.VMEM((16, 16), F32),
            pltpu.VMEM((16, 16), F32),
        ],
    )(x, b16, u, aggx_pp, aggx_pp, acc_ce_pp, acc_ce_pp,
      acc_re_pp, acc_re_pp,
      n1w1x, n1w1e, n1b1, n1w2, n1b2,
      n2w1x, n2w1h, n2w1u, n2b1, n2w2, n2b2,
      gw1u, gw1n, gw1e, gb1, gw2, gb2)


def kernel(x, edge_index, edge_attr, u, batch,
           ew1, eb1, ew2, eb2,
           n1w1, n1b1, n1w2, n1b2,
           n2w1, n2b1, n2w2, n2b2,
           gw1, gb1, gw2, gb2):
    n, dn = x.shape
    e, de = edge_attr.shape
    dg = u.shape[1]
    tn = 1000

    row = edge_index[0]
    col = edge_index[1]
    b16 = jnp.broadcast_to(batch[:, None], (n, 16))

    wxr = ew1[0:dn]
    wxc = ew1[dn:2 * dn]
    we = ew1[2 * dn:2 * dn + de]
    wu = ew1[2 * dn + de:]
    eb1r = eb1[None, :]
    eb2r = eb2[None, :]
    n1w1x = n1w1[0:dn]
    n1w1e = n1w1[dn:]
    n2w1x = n2w1[0:dn]
    n2w1h = n2w1[dn:dn + 128]
    n2w1u = n2w1[dn + 128:]
    gw1u = gw1[0:dg]
    gw1n = gw1[dg:dg + dn]
    gw1e = gw1[dg + dn:]

    a, c = _run_k1(x, b16, u, wxr, wxc, wu, eb1r, n, dn, tn)
    g = _run_k2(a, c, row, col, n, e)
    e_new = _run_k3(g, edge_attr, we, ew2, eb2r, e, de, 2000)
    e_view = jnp.reshape(e_new, (e // 8, 128))
    aggx_pp, acc_ce_pp, acc_re_pp = _run_k4(x, e_view, row, col, n, e)
    x_new, u_new = _run_k5(
        x, b16, u, aggx_pp, acc_ce_pp, acc_re_pp,
        n1w1x, n1w1e, n1b1[None, :], n1w2, n1b2[None, :],
        n2w1x, n2w1h, n2w1u, n2b1[None, :], n2w2, n2b2[None, :],
        gw1u, gw1n, gw1e, gb1[None, :], gw2, gb2[None, :],
        n, dn, dg, de, tn)
    return (x_new, e_new, u_new)


# speedup vs baseline: 7.6609x; 1.1653x over previous
import functools

import jax
import jax.numpy as jnp
from jax import lax
from jax.experimental import pallas as pl
from jax.experimental.pallas import tpu as pltpu
from jax.experimental.pallas import tpu_sc as plsc

F32 = jnp.float32

NC = 2
NS = 16
NW = NC * NS
LANES = 16

T_EDGE = 128
T_NODE = 80


def _k1_body(x_ref, b16_ref, u_ref, wxr_ref, wxc_ref, wu_ref, eb1_ref,
             a_ref, c_ref):
    x = x_ref[...]
    u2 = jnp.dot(u_ref[...], wu_ref[...], preferred_element_type=F32)
    tn = b16_ref.shape[0]
    iota = lax.broadcasted_iota(jnp.int32, (tn, 16), 1)
    oh = (b16_ref[...] == iota).astype(F32)
    a_ref[...] = (jnp.dot(x, wxr_ref[...], preferred_element_type=F32)
                  + jnp.dot(oh, u2, preferred_element_type=F32)
                  + eb1_ref[...])
    c_ref[...] = jnp.dot(x, wxc_ref[...], preferred_element_type=F32)


def _run_k1(x, b16, u, wxr, wxc, wu, eb1, n, dn, tn):
    nblk = n // tn
    full = lambda *shape: pl.BlockSpec(shape, lambda i: tuple(0 for _ in shape))
    return pl.pallas_call(
        _k1_body,
        grid=(nblk,),
        in_specs=[
            pl.BlockSpec((tn, dn), lambda i: (i, 0)),
            pl.BlockSpec((tn, 16), lambda i: (i, 0)),
            full(16, 32),
            full(dn, 128),
            full(dn, 128),
            full(32, 128),
            full(1, 128),
        ],
        out_specs=[
            pl.BlockSpec((tn, 128), lambda i: (i, 0)),
            pl.BlockSpec((tn, 128), lambda i: (i, 0)),
        ],
        out_shape=[
            jax.ShapeDtypeStruct((n, 128), F32),
            jax.ShapeDtypeStruct((n, 128), F32),
        ],
    )(x, b16, u, wxr, wxc, wu, eb1)


def _k2_body(n, e, a_hbm, c_hbm, row_hbm, col_hbm, g_hbm,
             idx_r, idx_c, buf_a, buf_c,
             sa0, sa1, sc0, sc1, sg0, sg1):
    cid = lax.axis_index("c")
    sid = lax.axis_index("s")
    wid = sid * NC + cid
    nchunks_tot = e // T_EDGE
    sa = (sa0, sa1)
    sc = (sc0, sc1)
    sg = (sg0, sg1)

    def load_idx(cc, slot):
        base = pl.multiple_of(cc * T_EDGE, T_EDGE)
        pltpu.sync_copy(row_hbm.at[pl.ds(base, T_EDGE)], idx_r.at[slot])
        pltpu.sync_copy(col_hbm.at[pl.ds(base, T_EDGE)], idx_c.at[slot])

    def fire_gathers(slot):
        pltpu.async_copy(a_hbm.at[idx_r.at[slot]], buf_a.at[slot], sa[slot])
        pltpu.async_copy(c_hbm.at[idx_c.at[slot]], buf_c.at[slot], sc[slot])

    def wait_gathers(slot):
        pltpu.make_async_copy(a_hbm.at[idx_r.at[slot]], buf_a.at[slot],
                              sa[slot]).wait()
        pltpu.make_async_copy(c_hbm.at[idx_c.at[slot]], buf_c.at[slot],
                              sc[slot]).wait()

    def wait_gwrite(cc, slot):
        base = pl.multiple_of(cc * T_EDGE, T_EDGE)
        pltpu.make_async_copy(buf_a.at[slot], g_hbm.at[pl.ds(base, T_EDGE)],
                              sg[slot]).wait()

    load_idx(wid, 0)
    fire_gathers(0)

    kmax = (nchunks_tot + NW - 1) // NW
    niter = (kmax + 1) // 2

    def step(ci, carry):
        for b in (0, 1):
            k = ci * 2 + b
            c = wid + k * NW
            cn = c + NW
            nb = 1 - b

            @pl.when(cn < nchunks_tot)
            def _():
                @pl.when(k >= 1)
                def _():
                    wait_gwrite(c - NW, nb)

                load_idx(cn, nb)
                fire_gathers(nb)

            @pl.when(c < nchunks_tot)
            def _():
                wait_gathers(b)
                base = pl.multiple_of(c * T_EDGE, T_EDGE)

                def addrow(i, cy):
                    for j in range(128 // LANES):
                        sl = pl.ds(j * LANES, LANES)
                        buf_a[b, i, sl] = buf_a[b, i, sl] + buf_c[b, i, sl]
                    return cy

                lax.fori_loop(0, T_EDGE, addrow, 0)
                pltpu.async_copy(buf_a.at[b], g_hbm.at[pl.ds(base, T_EDGE)],
                                 sg[b])
        return carry

    lax.fori_loop(0, niter, step, 0)
    nchunks_w = (nchunks_tot - wid + NW - 1) // NW
    for b in (0, 1):
        kb = nchunks_w - 1 - ((nchunks_w - 1 - b) % 2)
        wait_gwrite(wid + kb * NW, b)


def _run_k2(a, c, row, col, n, e):
    mesh = plsc.VectorSubcoreMesh(core_axis_name="c", subcore_axis_name="s",
                                  num_cores=NC, num_subcores=NS)
    k = functools.partial(
        pl.kernel,
        out_type=jax.ShapeDtypeStruct((e, 128), F32),
        mesh=mesh,
        scratch_types=[
            pltpu.VMEM((2, T_EDGE), jnp.int32),
            pltpu.VMEM((2, T_EDGE), jnp.int32),
            pltpu.VMEM((2, T_EDGE, 128), F32),
            pltpu.VMEM((2, T_EDGE, 128), F32),
            pltpu.SemaphoreType.DMA,
            pltpu.SemaphoreType.DMA,
            pltpu.SemaphoreType.DMA,
            pltpu.SemaphoreType.DMA,
            pltpu.SemaphoreType.DMA,
            pltpu.SemaphoreType.DMA,
        ],
    )(functools.partial(_k2_body, n, e))
    return k(a, c, row, col)


def _k3_body(g_ref, ea_ref, we_ref, ew2_ref, eb2_ref, e_ref):
    eh = jnp.maximum(
        g_ref[...] + jnp.dot(ea_ref[...], we_ref[...],
                             preferred_element_type=F32), 0.0)
    e_ref[...] = (jnp.dot(eh, ew2_ref[...], preferred_element_type=F32)
                  + eb2_ref[...])


def _run_k3(g, ea, we, ew2, eb2, e, de, te):
    nblk = e // te
    full = lambda *shape: pl.BlockSpec(shape, lambda i: tuple(0 for _ in shape))
    return pl.pallas_call(
        _k3_body,
        grid=(nblk,),
        in_specs=[
            pl.BlockSpec((te, 128), lambda i: (i, 0)),
            pl.BlockSpec((te, de), lambda i: (i, 0)),
            full(de, 128),
            full(128, de),
            full(1, de),
        ],
        out_specs=pl.BlockSpec((te, de), lambda i: (i, 0)),
        out_shape=jax.ShapeDtypeStruct((e, de), F32),
    )(g, ea, we, ew2, eb2)


def _k4a_body(n, e, x_hbm, row_hbm, col_hbm, aggx_hbm,
              idxa, idxb, buf_s, zbuf, acc_sh,
              si0, si1, ss0, ss1):
    cid = lax.axis_index("c")
    sid = lax.axis_index("s")
    wid = sid * NC + cid
    nchunks_tot = e // T_EDGE
    nchunks_n = n // T_NODE
    si = (si0, si1)
    ss = (ss0, ss1)

    zv = jnp.zeros((LANES,), F32)
    nz_w = (nchunks_n - sid + NS - 1) // NS
    kmax = (nchunks_tot + NW - 1) // NW
    niter = (kmax + 1) // 2

    def z0(i, carry):
        for j in range(128 // LANES):
            zbuf[i, pl.ds(j * LANES, LANES)] = zv
        return carry

    lax.fori_loop(0, T_NODE, z0, 0)

    def zchunk(ci, carry):
        b = pl.multiple_of((sid + ci * NS) * T_NODE, T_NODE)
        pltpu.sync_copy(zbuf, acc_sh.at[pl.ds(b, T_NODE)])
        return carry

    lax.fori_loop(0, nz_w, zchunk, 0)
    plsc.subcore_barrier()

    def wait_scatter(slot):
        pltpu.make_async_copy(buf_s.at[slot], acc_sh.at[idxb.at[slot]],
                              ss[slot]).wait()

    def load_idx_x(cc, slot):
        base = pl.multiple_of(cc * T_EDGE, T_EDGE)
        pltpu.sync_copy(row_hbm.at[pl.ds(base, T_EDGE)], idxa.at[slot])
        pltpu.sync_copy(col_hbm.at[pl.ds(base, T_EDGE)], idxb.at[slot])

    load_idx_x(wid, 0)
    pltpu.async_copy(x_hbm.at[idxa.at[0]], buf_s.at[0], si[0])

    def step_x(ci, carry):
        for b in (0, 1):
            k = ci * 2 + b
            c = wid + k * NW
            cn = c + NW
            nb = 1 - b

            @pl.when(cn < nchunks_tot)
            def _():
                @pl.when(k >= 1)
                def _():
                    wait_scatter(nb)

                load_idx_x(cn, nb)
                pltpu.async_copy(x_hbm.at[idxa.at[nb]], buf_s.at[nb],
                                 si[nb])

            @pl.when(c < nchunks_tot)
            def _():
                pltpu.make_async_copy(x_hbm.at[idxa.at[b]], buf_s.at[b],
                                      si[b]).wait()
                pltpu.async_copy(buf_s.at[b], acc_sh.at[idxb.at[b]],
                                 ss[b], add=True)
        return carry

    lax.fori_loop(0, niter, step_x, 0)
    for b in (0, 1):
        wait_scatter(b)
    plsc.subcore_barrier()

    def ochunk(ci, carry):
        b = pl.multiple_of((sid + ci * NS) * T_NODE, T_NODE)
        ob = pl.multiple_of(cid * n + (sid + ci * NS) * T_NODE, T_NODE)
        pltpu.sync_copy(acc_sh.at[pl.ds(b, T_NODE)],
                        aggx_hbm.at[pl.ds(ob, T_NODE)])
        return carry

    lax.fori_loop(0, nz_w, ochunk, 0)


def _run_k4a(x, row, col, n, e):
    mesh = plsc.VectorSubcoreMesh(core_axis_name="c", subcore_axis_name="s",
                                  num_cores=NC, num_subcores=NS)
    k = functools.partial(
        pl.kernel,
        out_type=jax.ShapeDtypeStruct((NC * n, 128), F32),
        mesh=mesh,
        scratch_types=[
            pltpu.VMEM((2, T_EDGE), jnp.int32),
            pltpu.VMEM((2, T_EDGE), jnp.int32),
            pltpu.VMEM((2, T_EDGE, 128), F32),
            pltpu.VMEM((T_NODE, 128), F32),
            pltpu.VMEM_SHARED((n, 128), F32),
            pltpu.SemaphoreType.DMA,
            pltpu.SemaphoreType.DMA,
            pltpu.SemaphoreType.DMA,
            pltpu.SemaphoreType.DMA,
        ],
    )(functools.partial(_k4a_body, n, e))
    return k(x, row, col)


def _k4_body(n, e, ev_hbm, row_hbm, col_hbm,
             acc_ce_hbm, acc_re_hbm,
             idxa, buf_e, buf_s, zbuf, acc_sh,
             si0, si1, ss0, ss1):
    cid = lax.axis_index("c")
    sid = lax.axis_index("s")
    wid = sid * NC + cid
    nchunks_tot = e // T_EDGE
    nchunks_n = n // T_NODE
    erows = T_EDGE // 8
    si = (si0, si1)
    ss = (ss0, ss1)

    zv = jnp.zeros((LANES,), F32)
    lane = lax.iota(jnp.int32, LANES)
    onev = jnp.where(lane == 0, 1.0, 0.0).astype(F32)

    nz_w = (nchunks_n - sid + NS - 1) // NS
    kmax = (nchunks_tot + NW - 1) // NW
    niter = (kmax + 1) // 2
    nchunks_w = (nchunks_tot - wid + NW - 1) // NW

    def z0(i, carry):
        for j in range(128 // LANES):
            zbuf[i, pl.ds(j * LANES, LANES)] = zv
        return carry

    lax.fori_loop(0, T_NODE, z0, 0)

    def z1(i, carry):
        for b in (0, 1):
            for j in range(128 // LANES):
                buf_s[b, i, pl.ds(j * LANES, LANES)] = zv
        return carry

    lax.fori_loop(0, T_EDGE, z1, 0)

    def zero_acc():
        def zchunk(ci, carry):
            b = pl.multiple_of((sid + ci * NS) * T_NODE, T_NODE)
            pltpu.sync_copy(zbuf, acc_sh.at[pl.ds(b, T_NODE)])
            return carry

        lax.fori_loop(0, nz_w, zchunk, 0)

    def copy_out(out_hbm):
        def ochunk(ci, carry):
            b = pl.multiple_of((sid + ci * NS) * T_NODE, T_NODE)
            ob = pl.multiple_of(cid * n + (sid + ci * NS) * T_NODE, T_NODE)
            pltpu.sync_copy(acc_sh.at[pl.ds(b, T_NODE)],
                            out_hbm.at[pl.ds(ob, T_NODE)])
            return carry

        lax.fori_loop(0, nz_w, ochunk, 0)

    def wait_scatter(slot, sidx_ref):
        pltpu.make_async_copy(buf_s.at[slot], acc_sh.at[sidx_ref.at[slot]],
                              ss[slot]).wait()

    def drain_tail(sidx_ref):
        for b in (0, 1):
            wait_scatter(b, sidx_ref)

    for phase, (idx_hbm, out_hbm, markv) in enumerate(
            ((col_hbm, acc_ce_hbm, zv), (row_hbm, acc_re_hbm, onev))):
        zero_acc()
        plsc.subcore_barrier()

        def fire_reads(cc, slot):
            base = pl.multiple_of(cc * T_EDGE, T_EDGE)
            vbase = pl.multiple_of(cc * erows, erows)
            pltpu.async_copy(idx_hbm.at[pl.ds(base, T_EDGE)],
                             idxa.at[slot], si[slot])
            pltpu.async_copy(ev_hbm.at[pl.ds(vbase, erows)],
                             buf_e.at[slot], si[slot])

        def wait_reads(cc, slot):
            base = pl.multiple_of(cc * T_EDGE, T_EDGE)
            vbase = pl.multiple_of(cc * erows, erows)
            pltpu.make_async_copy(idx_hbm.at[pl.ds(base, T_EDGE)],
                                  idxa.at[slot], si[slot]).wait()
            pltpu.make_async_copy(ev_hbm.at[pl.ds(vbase, erows)],
                                  buf_e.at[slot], si[slot]).wait()

        fire_reads(wid, 0)

        def step(ci, carry):
            for b in (0, 1):
                k = ci * 2 + b
                c = wid + k * NW
                cn = c + NW
                nb = 1 - b

                @pl.when(cn < nchunks_tot)
                def _():
                    @pl.when(k >= 1)
                    def _():
                        wait_scatter(nb, idxa)

                    fire_reads(cn, nb)

                @pl.when(c < nchunks_tot)
                def _():
                    wait_reads(c, b)

                    def build(i, cy):
                        for j in range(8):
                            v = buf_e[b, i, pl.ds(j * LANES, LANES)]
                            buf_s[b, i * 8 + j, pl.ds(0, LANES)] = v
                            buf_s[b, i * 8 + j, pl.ds(LANES, LANES)] = markv
                        return cy

                    lax.fori_loop(0, erows, build, 0)
                    pltpu.async_copy(buf_s.at[b], acc_sh.at[idxa.at[b]],
                                     ss[b], add=True)
            return carry

        lax.fori_loop(0, niter, step, 0)
        drain_tail(idxa)
        plsc.subcore_barrier()
        copy_out(out_hbm)
        if phase == 0:
            plsc.subcore_barrier()


def _run_k4(e_view, row, col, n, e):
    mesh = plsc.VectorSubcoreMesh(core_axis_name="c", subcore_axis_name="s",
                                  num_cores=NC, num_subcores=NS)
    sds = jax.ShapeDtypeStruct((NC * n, 128), F32)
    k = functools.partial(
        pl.kernel,
        out_type=(sds, sds),
        mesh=mesh,
        scratch_types=[
            pltpu.VMEM((2, T_EDGE), jnp.int32),
            pltpu.VMEM((2, T_EDGE // 8, 128), F32),
            pltpu.VMEM((2, T_EDGE, 128), F32),
            pltpu.VMEM((T_NODE, 128), F32),
            pltpu.VMEM_SHARED((n, 128), F32),
            pltpu.SemaphoreType.DMA,
            pltpu.SemaphoreType.DMA,
            pltpu.SemaphoreType.DMA,
            pltpu.SemaphoreType.DMA,
        ],
    )(functools.partial(_k4_body, n, e))
    return k(e_view, row, col)


def _k5_body(nblk,
             x_ref, b16_ref, u_ref,
             aggx0, aggx1, ce0, ce1, re0, re1,
             n1w1x, n1w1e, n1b1, n1w2, n1b2,
             n2w1x, n2w1h, n2w1u, n2b1, n2w2, n2b2,
             gw1u, gw1n, gw1e, gb1, gw2, gb2,
             xn_ref, un_ref,
             s_nsum, s_esum, s_ecnt, s_ncnt):
    i = pl.program_id(0)
    tn = x_ref.shape[0]
    iota = lax.broadcasted_iota(jnp.int32, (tn, 16), 1)
    oh = (b16_ref[...] == iota).astype(F32)

    aggx = aggx0[...] + aggx1[...]
    acc_ce = ce0[...] + ce1[...]
    acc_re = re0[...] + re1[...]
    aggec = acc_ce[:, 0:16]
    agger = acc_re[:, 0:16]
    cntcol = acc_re[:, 16:32]

    s = (jnp.dot(aggx, n1w1x[...], preferred_element_type=F32)
         + jnp.dot(aggec, n1w1e[...], preferred_element_type=F32)
         + n1b1[...])
    h = (jnp.dot(jnp.maximum(s, 0.0), n1w2[...], preferred_element_type=F32)
         + n1b2[...])
    u3 = jnp.dot(u_ref[...], n2w1u[...], preferred_element_type=F32)
    pre = (jnp.dot(x_ref[...], n2w1x[...], preferred_element_type=F32)
           + jnp.dot(h, n2w1h[...], preferred_element_type=F32)
           + jnp.dot(oh, u3, preferred_element_type=F32)
           + n2b1[...])
    xn = (jnp.dot(jnp.maximum(pre, 0.0), n2w2[...],
                  preferred_element_type=F32) + n2b2[...])
    xn_ref[...] = xn

    @pl.when(i == 0)
    def _():
        s_nsum[...] = jnp.zeros_like(s_nsum)
        s_esum[...] = jnp.zeros_like(s_esum)
        s_ecnt[...] = jnp.zeros_like(s_ecnt)
        s_ncnt[...] = jnp.zeros_like(s_ncnt)

    dnums = (((0,), (0,)), ((), ()))
    s_nsum[...] += lax.dot_general(oh, xn, dnums, preferred_element_type=F32)
    s_esum[...] += lax.dot_general(oh, agger, dnums,
                                   preferred_element_type=F32)
    s_ecnt[...] += lax.dot_general(oh, cntcol, dnums,
                                   preferred_element_type=F32)
    s_ncnt[...] += lax.dot_general(oh, jnp.ones_like(oh), dnums,
                                   preferred_element_type=F32)

    @pl.when(i == nblk - 1)
    def _():
        ncnt = jnp.maximum(s_ncnt[:, 0:1], 1.0)
        node_info = s_nsum[...] / ncnt
        ecnt = jnp.maximum(s_ecnt[:, 0:1], 1.0)
        edge_info = s_esum[...] / ecnt
        g1 = (jnp.dot(u_ref[...], gw1u[...], preferred_element_type=F32)
              + jnp.dot(node_info, gw1n[...], preferred_element_type=F32)
              + jnp.dot(edge_info, gw1e[...], preferred_element_type=F32)
              + gb1[...])
        un_ref[...] = (jnp.dot(jnp.maximum(g1, 0.0), gw2[...],
                               preferred_element_type=F32) + gb2[...])


def _run_k5(x, b16, u, aggx_pp, acc_ce_pp, acc_re_pp,
            n1w1x, n1w1e, n1b1, n1w2, n1b2,
            n2w1x, n2w1h, n2w1u, n2b1, n2w2, n2b2,
            gw1u, gw1n, gw1e, gb1, gw2, gb2,
            n, dn, dg, de, tn):
    nblk = n // tn
    full = lambda *shape: pl.BlockSpec(shape, lambda i: tuple(0 for _ in shape))
    p0 = pl.BlockSpec((tn, 128), lambda i: (i, 0))
    p1 = pl.BlockSpec((tn, 128), lambda i: (i + nblk, 0))
    return pl.pallas_call(
        functools.partial(_k5_body, nblk),
        grid=(nblk,),
        in_specs=[
            pl.BlockSpec((tn, dn), lambda i: (i, 0)),
            pl.BlockSpec((tn, 16), lambda i: (i, 0)),
            full(16, dg),
            p0, p1, p0, p1, p0, p1,
            full(dn, 128), full(de, 128), full(1, 128),
            full(128, 128), full(1, 128),
            full(dn, 128), full(128, 128), full(dg, 128), full(1, 128),
            full(128, dn), full(1, dn),
            full(dg, 128), full(dn, 128), full(de, 128), full(1, 128),
            full(128, dg), full(1, dg),
        ],
        out_specs=[
            pl.BlockSpec((tn, dn), lambda i: (i, 0)),
            pl.BlockSpec((16, dg), lambda i: (0, 0)),
        ],
        out_shape=[
            jax.ShapeDtypeStruct((n, dn), F32),
            jax.ShapeDtypeStruct((16, dg), F32),
        ],
        scratch_shapes=[
            pltpu.VMEM((16, 128), F32),
            pltpu.VMEM((16, de), F32),
            pltpu.VMEM((16, 16), F32),
            pltpu.VMEM((16, 16), F32),
        ],
    )(x, b16, u, aggx_pp, aggx_pp, acc_ce_pp, acc_ce_pp,
      acc_re_pp, acc_re_pp,
      n1w1x, n1w1e, n1b1, n1w2, n1b2,
      n2w1x, n2w1h, n2w1u, n2b1, n2w2, n2b2,
      gw1u, gw1n, gw1e, gb1, gw2, gb2)


def kernel(x, edge_index, edge_attr, u, batch,
           ew1, eb1, ew2, eb2,
           n1w1, n1b1, n1w2, n1b2,
           n2w1, n2b1, n2w2, n2b2,
           gw1, gb1, gw2, gb2):
    n, dn = x.shape
    e, de = edge_attr.shape
    dg = u.shape[1]
    tn = 1000

    row = edge_index[0]
    col = edge_index[1]
    b16 = jnp.broadcast_to(batch[:, None], (n, 16))

    wxr = ew1[0:dn]
    wxc = ew1[dn:2 * dn]
    we = ew1[2 * dn:2 * dn + de]
    wu = ew1[2 * dn + de:]
    eb1r = eb1[None, :]
    eb2r = eb2[None, :]
    n1w1x = n1w1[0:dn]
    n1w1e = n1w1[dn:]
    n2w1x = n2w1[0:dn]
    n2w1h = n2w1[dn:dn + 128]
    n2w1u = n2w1[dn + 128:]
    gw1u = gw1[0:dg]
    gw1n = gw1[dg:dg + dn]
    gw1e = gw1[dg + dn:]

    aggx_pp = _run_k4a(x, row, col, n, e)
    a, c = _run_k1(x, b16, u, wxr, wxc, wu, eb1r, n, dn, tn)
    g = _run_k2(a, c, row, col, n, e)
    e_new = _run_k3(g, edge_attr, we, ew2, eb2r, e, de, 2000)
    e_view = jnp.reshape(e_new, (e // 8, 128))
    acc_ce_pp, acc_re_pp = _run_k4(e_view, row, col, n, e)
    x_new, u_new = _run_k5(
        x, b16, u, aggx_pp, acc_ce_pp, acc_re_pp,
        n1w1x, n1w1e, n1b1[None, :], n1w2, n1b2[None, :],
        n2w1x, n2w1h, n2w1u, n2b1[None, :], n2w2, n2b2[None, :],
        gw1u, gw1n, gw1e, gb1[None, :], gw2, gb2[None, :],
        n, dn, dg, de, tn)
    return (x_new, e_new, u_new)
